# baseline (speedup 1.0000x reference)
import jax
import jax.numpy as jnp
from jax.experimental import pallas as pl
from jax.experimental.pallas import tpu as pltpu

_NUM_EXPERTS = 8
_HIDDEN = 768
_BLOCK = 1024


def _router_block(x_ref, w_ref, b_ref, logits_ref, weights_ref, index_ref):
    x = x_ref[...]
    w = w_ref[...]
    b = b_ref[...]
    logits = jnp.dot(x, w, preferred_element_type=jnp.float32) + b
    logits_ref[...] = logits

    lt = logits.T

    m = jnp.max(lt, axis=0, keepdims=True)
    ex = jnp.exp(lt - m)
    s = jnp.sum(ex, axis=0, keepdims=True)

    iota = jax.lax.broadcasted_iota(jnp.int32, lt.shape, 0)
    sentinel = jnp.int32(_NUM_EXPERTS)
    idx1 = jnp.min(jnp.where(lt == m, iota, sentinel), axis=0, keepdims=True)
    masked = jnp.where(iota == idx1, -jnp.inf, lt)
    m2 = jnp.max(masked, axis=0, keepdims=True)
    idx2 = jnp.min(jnp.where(masked == m2, iota, sentinel), axis=0, keepdims=True)

    w1 = jnp.ones_like(s) / s
    w2 = jnp.exp(m2 - m) / s
    weights_ref[...] = jnp.concatenate([w1, w2], axis=0).T
    index_ref[...] = jnp.concatenate([idx1, idx2], axis=0).T


def kernel(hidden_states, W, b):
    batch, seq, hidden = hidden_states.shape
    n_tokens = batch * seq
    x = hidden_states.reshape(n_tokens, hidden)
    wt = W.T
    b2 = b.reshape(1, _NUM_EXPERTS)

    grid = (n_tokens // _BLOCK,)
    logits, weights, index = pl.pallas_call(
        _router_block,
        grid=grid,
        in_specs=[
            pl.BlockSpec((_BLOCK, hidden), lambda i: (i, 0)),
            pl.BlockSpec((hidden, _NUM_EXPERTS), lambda i: (0, 0)),
            pl.BlockSpec((1, _NUM_EXPERTS), lambda i: (0, 0)),
        ],
        out_specs=[
            pl.BlockSpec((_BLOCK, _NUM_EXPERTS), lambda i: (i, 0)),
            pl.BlockSpec((_BLOCK, 2), lambda i: (i, 0)),
            pl.BlockSpec((_BLOCK, 2), lambda i: (i, 0)),
        ],
        out_shape=[
            jax.ShapeDtypeStruct((n_tokens, _NUM_EXPERTS), jnp.float32),
            jax.ShapeDtypeStruct((n_tokens, 2), jnp.float32),
            jax.ShapeDtypeStruct((n_tokens, 2), jnp.int32),
        ],
        compiler_params=pltpu.CompilerParams(
            dimension_semantics=("parallel",),
        ),
    )(x, wt, b2)

    return (
        index.reshape(-1),
        weights.reshape(batch, seq, 2),
        logits.reshape(batch, seq, _NUM_EXPERTS),
    )


# speedup vs baseline: 1.1267x; 1.1267x over previous
import jax
import jax.numpy as jnp
from jax.experimental import pallas as pl
from jax.experimental.pallas import tpu as pltpu

_NUM_EXPERTS = 8
_HIDDEN = 768
_BLOCK = 4096


def _router_block(x_ref, w_ref, b_ref, logits_ref, weights_ref, index_ref):
    x = x_ref[...]
    w = w_ref[...]
    b = b_ref[...]
    logits = jnp.dot(x, w, preferred_element_type=jnp.float32) + b
    logits_ref[...] = logits

    lt = logits.T

    m = jnp.max(lt, axis=0, keepdims=True)
    ex = jnp.exp(lt - m)
    s = jnp.sum(ex, axis=0, keepdims=True)

    iota = jax.lax.broadcasted_iota(jnp.int32, lt.shape, 0)
    sentinel = jnp.int32(_NUM_EXPERTS)
    idx1 = jnp.min(jnp.where(lt == m, iota, sentinel), axis=0, keepdims=True)
    masked = jnp.where(iota == idx1, -jnp.inf, lt)
    m2 = jnp.max(masked, axis=0, keepdims=True)
    idx2 = jnp.min(jnp.where(masked == m2, iota, sentinel), axis=0, keepdims=True)

    w1 = jnp.ones_like(s) / s
    w2 = jnp.exp(m2 - m) / s
    weights_ref[...] = jnp.concatenate([w1, w2], axis=0).T
    index_ref[...] = jnp.concatenate([idx1, idx2], axis=0).T


def kernel(hidden_states, W, b):
    batch, seq, hidden = hidden_states.shape
    n_tokens = batch * seq
    x = hidden_states.reshape(n_tokens, hidden)
    wt = W.T
    b2 = b.reshape(1, _NUM_EXPERTS)

    grid = (n_tokens // _BLOCK,)
    logits, weights, index = pl.pallas_call(
        _router_block,
        grid=grid,
        in_specs=[
            pl.BlockSpec((_BLOCK, hidden), lambda i: (i, 0)),
            pl.BlockSpec((hidden, _NUM_EXPERTS), lambda i: (0, 0)),
            pl.BlockSpec((1, _NUM_EXPERTS), lambda i: (0, 0)),
        ],
        out_specs=[
            pl.BlockSpec((_BLOCK, _NUM_EXPERTS), lambda i: (i, 0)),
            pl.BlockSpec((_BLOCK, 2), lambda i: (i, 0)),
            pl.BlockSpec((_BLOCK, 2), lambda i: (i, 0)),
        ],
        out_shape=[
            jax.ShapeDtypeStruct((n_tokens, _NUM_EXPERTS), jnp.float32),
            jax.ShapeDtypeStruct((n_tokens, 2), jnp.float32),
            jax.ShapeDtypeStruct((n_tokens, 2), jnp.int32),
        ],
        compiler_params=pltpu.CompilerParams(
            dimension_semantics=("parallel",),
        ),
    )(x, wt, b2)

    return (
        index.reshape(-1),
        weights.reshape(batch, seq, 2),
        logits.reshape(batch, seq, _NUM_EXPERTS),
    )


# speedup vs baseline: 1.1362x; 1.0084x over previous
import jax
import jax.numpy as jnp
from jax.experimental import pallas as pl
from jax.experimental.pallas import tpu as pltpu

_NUM_EXPERTS = 8
_HIDDEN = 768
_BLOCK = 2048
_NSLICE = 2


def _route_slice(x, w, b, j, logits_ref, weights_ref, index_ref):
    logits = jnp.dot(x, w, preferred_element_type=jnp.float32) + b
    logits_ref[pl.ds(j * _BLOCK, _BLOCK), :] = logits

    lt = logits.T

    m = jnp.max(lt, axis=0, keepdims=True)
    ex = jnp.exp(lt - m)
    s = jnp.sum(ex, axis=0, keepdims=True)

    iota = jax.lax.broadcasted_iota(jnp.int32, lt.shape, 0)
    sentinel = jnp.int32(_NUM_EXPERTS)
    idx1 = jnp.min(jnp.where(lt == m, iota, sentinel), axis=0, keepdims=True)
    masked = jnp.where(iota == idx1, -jnp.inf, lt)
    m2 = jnp.max(masked, axis=0, keepdims=True)
    idx2 = jnp.min(jnp.where(masked == m2, iota, sentinel), axis=0, keepdims=True)

    w1 = jnp.ones_like(s) / s
    w2 = jnp.exp(m2 - m) / s
    weights_ref[pl.ds(j * _BLOCK, _BLOCK), :] = jnp.concatenate([w1, w2], axis=0).T
    index_ref[pl.ds(j * _BLOCK, _BLOCK), :] = jnp.concatenate([idx1, idx2], axis=0).T


def _router_block(*refs):
    s = _NSLICE
    xs = refs[:s]
    w = refs[s][...]
    b = refs[s + 1][...]
    logits_ref, weights_ref, index_ref = refs[s + 2:s + 5]
    for j in range(s):
        _route_slice(xs[j][...], w, b, j, logits_ref, weights_ref, index_ref)


def kernel(hidden_states, W, b):
    batch, seq, hidden = hidden_states.shape
    n_tokens = batch * seq
    x = hidden_states.reshape(n_tokens, hidden)
    wt = W.T
    b2 = b.reshape(1, _NUM_EXPERTS)

    s = _NSLICE
    step = _BLOCK * s
    grid = (n_tokens // step,)

    def slice_map(j):
        return lambda i: (i * s + j, 0)

    in_specs = [pl.BlockSpec((_BLOCK, hidden), slice_map(j)) for j in range(s)]
    in_specs += [
        pl.BlockSpec((hidden, _NUM_EXPERTS), lambda i: (0, 0)),
        pl.BlockSpec((1, _NUM_EXPERTS), lambda i: (0, 0)),
    ]
    out_specs = [
        pl.BlockSpec((step, _NUM_EXPERTS), lambda i: (i, 0)),
        pl.BlockSpec((step, 2), lambda i: (i, 0)),
        pl.BlockSpec((step, 2), lambda i: (i, 0)),
    ]
    out_shape = [
        jax.ShapeDtypeStruct((n_tokens, _NUM_EXPERTS), jnp.float32),
        jax.ShapeDtypeStruct((n_tokens, 2), jnp.float32),
        jax.ShapeDtypeStruct((n_tokens, 2), jnp.int32),
    ]
    logits, weights, index = pl.pallas_call(
        _router_block,
        grid=grid,
        in_specs=in_specs,
        out_specs=out_specs,
        out_shape=out_shape,
        compiler_params=pltpu.CompilerParams(
            dimension_semantics=("arbitrary",),
        ),
    )(*([x] * s), wt, b2)

    return (
        index.reshape(-1),
        weights.reshape(batch, seq, 2),
        logits.reshape(batch, seq, _NUM_EXPERTS),
    )


# speedup vs baseline: 1.1467x; 1.0092x over previous
import jax
import jax.numpy as jnp
from jax.experimental import pallas as pl
from jax.experimental.pallas import tpu as pltpu

_NUM_EXPERTS = 8
_HIDDEN = 768
_BLOCK = 1024
_NSLICE = 4


def _route_slice(x, w, b, j, logits_ref, weights_ref, index_ref):
    logits = jnp.dot(x, w, preferred_element_type=jnp.float32) + b
    logits_ref[pl.ds(j * _BLOCK, _BLOCK), :] = logits

    lt = logits.T

    m = jnp.max(lt, axis=0, keepdims=True)
    ex = jnp.exp(lt - m)
    s = jnp.sum(ex, axis=0, keepdims=True)

    iota = jax.lax.broadcasted_iota(jnp.int32, lt.shape, 0)
    sentinel = jnp.int32(_NUM_EXPERTS)
    idx1 = jnp.min(jnp.where(lt == m, iota, sentinel), axis=0, keepdims=True)
    masked = jnp.where(iota == idx1, -jnp.inf, lt)
    m2 = jnp.max(masked, axis=0, keepdims=True)
    idx2 = jnp.min(jnp.where(masked == m2, iota, sentinel), axis=0, keepdims=True)

    w1 = jnp.ones_like(s) / s
    w2 = jnp.exp(m2 - m) / s
    weights_ref[pl.ds(j * _BLOCK, _BLOCK), :] = jnp.concatenate([w1, w2], axis=0).T
    index_ref[pl.ds(j * _BLOCK, _BLOCK), :] = jnp.concatenate([idx1, idx2], axis=0).T


def _router_block(*refs):
    s = _NSLICE
    xs = refs[:s]
    w = refs[s][...]
    b = refs[s + 1][...]
    logits_ref, weights_ref, index_ref = refs[s + 2:s + 5]
    for j in range(s):
        _route_slice(xs[j][...], w, b, j, logits_ref, weights_ref, index_ref)


def kernel(hidden_states, W, b):
    batch, seq, hidden = hidden_states.shape
    n_tokens = batch * seq
    x = hidden_states.reshape(n_tokens, hidden)
    wt = W.T
    b2 = b.reshape(1, _NUM_EXPERTS)

    s = _NSLICE
    step = _BLOCK * s
    grid = (n_tokens // step,)

    def slice_map(j):
        return lambda i: (i * s + j, 0)

    in_specs = [pl.BlockSpec((_BLOCK, hidden), slice_map(j)) for j in range(s)]
    in_specs += [
        pl.BlockSpec((hidden, _NUM_EXPERTS), lambda i: (0, 0)),
        pl.BlockSpec((1, _NUM_EXPERTS), lambda i: (0, 0)),
    ]
    out_specs = [
        pl.BlockSpec((step, _NUM_EXPERTS), lambda i: (i, 0)),
        pl.BlockSpec((step, 2), lambda i: (i, 0)),
        pl.BlockSpec((step, 2), lambda i: (i, 0)),
    ]
    out_shape = [
        jax.ShapeDtypeStruct((n_tokens, _NUM_EXPERTS), jnp.float32),
        jax.ShapeDtypeStruct((n_tokens, 2), jnp.float32),
        jax.ShapeDtypeStruct((n_tokens, 2), jnp.int32),
    ]
    logits, weights, index = pl.pallas_call(
        _router_block,
        grid=grid,
        in_specs=in_specs,
        out_specs=out_specs,
        out_shape=out_shape,
        compiler_params=pltpu.CompilerParams(
            dimension_semantics=("arbitrary",),
        ),
    )(*([x] * s), wt, b2)

    return (
        index.reshape(-1),
        weights.reshape(batch, seq, 2),
        logits.reshape(batch, seq, _NUM_EXPERTS),
    )
